# baseline (speedup 1.0000x reference)
import functools

import jax
import jax.numpy as jnp
from jax import lax
from jax.experimental import pallas as pl
from jax.experimental.pallas import tpu as pltpu
from jax.experimental.pallas import tpu_sc as plsc

ROWS = 16384
COLS = 200
L = 16
VOCAB_PAD = 32
NC, NS = 2, 16
NW = NC * NS
ROWS_PER_W = ROWS // NW
CHUNK = 128
NCHUNK = ROWS_PER_W // CHUNK
FULL = COLS // L
TAIL_OFF = COLS - L
TAIL_DUP = FULL * L - TAIL_OFF

_mesh = plsc.VectorSubcoreMesh(
    core_axis_name="c", subcore_axis_name="s", num_cores=NC, num_subcores=NS
)


@functools.partial(
    pl.kernel,
    out_type=jax.ShapeDtypeStruct((ROWS,), jnp.float32),
    mesh=_mesh,
    compiler_params=pltpu.CompilerParams(needs_layout_passes=False),
    scratch_types=[
        pltpu.VMEM((CHUNK * COLS,), jnp.int32),
        pltpu.VMEM((CHUNK * COLS,), jnp.int32),
        pltpu.VMEM((VOCAB_PAD, L), jnp.float32),
        pltpu.VMEM((ROWS_PER_W,), jnp.float32),
        pltpu.SemaphoreType.DMA,
        pltpu.SemaphoreType.DMA,
    ],
)
def _sc_kernel(seq_hbm, tab_hbm, out_hbm, seq_v0, seq_v1, tab_v, out_v, sem0, sem1):
    wid = lax.axis_index("s") * NC + lax.axis_index("c")
    row0 = wid * ROWS_PER_W

    pltpu.sync_copy(tab_hbm, tab_v)

    sems = (sem0, sem1)

    bufs = (seq_v0, seq_v1)

    def start(c, buf):
        return pltpu.async_copy(
            seq_hbm.at[pl.ds((row0 + c * CHUNK) * COLS, CHUNK * COLS)],
            bufs[buf],
            sems[buf],
        )

    pending = start(0, 0)
    for c in range(NCHUNK):
        buf = c % 2
        pending.wait()
        if c + 1 < NCHUNK:
            pending = start(c + 1, 1 - buf)
        sv = bufs[buf]

        def pair_body(g, _, sv=sv, c=c):
            base = g * (2 * COLS)
            lanes = lax.iota(jnp.int32, L)
            low_keep = (lanes < TAIL_DUP).astype(jnp.float32)
            high_keep = jnp.float32(1.0) - low_keep
            last_lane = lanes == L - 1
            acc_a = jnp.zeros((L,), jnp.float32)
            acc_b = jnp.zeros((L,), jnp.float32)
            for k in range(FULL):
                v = sv[pl.ds(base + k * L, L)]
                acc_a = acc_a + plsc.load_gather(tab_v, [v, lanes])
            v = sv[pl.ds(base + FULL * L, L)]
            mid = plsc.load_gather(tab_v, [v, lanes])
            acc_a = acc_a + mid * low_keep
            acc_b = acc_b + mid * high_keep
            for k in range(FULL + 1, 2 * FULL + 1):
                v = sv[pl.ds(base + k * L, L)]
                acc_b = acc_b + plsc.load_gather(tab_v, [v, lanes])
            row = c * CHUNK + 2 * g
            plsc.store_scatter(
                out_v, [jnp.full((L,), row, jnp.int32)],
                plsc.cumsum(acc_a), mask=last_lane)
            plsc.store_scatter(
                out_v, [jnp.full((L,), row + 1, jnp.int32)],
                plsc.cumsum(acc_b), mask=last_lane)
            return 0

        lax.fori_loop(0, CHUNK // 2, pair_body, 0)

    inv = jnp.float32(1.0 / COLS)
    for i in range(ROWS_PER_W // L):
        out_v[pl.ds(i * L, L)] = out_v[pl.ds(i * L, L)] * inv

    pltpu.sync_copy(out_v, out_hbm.at[pl.ds(row0, ROWS_PER_W)])


def kernel(seq_samples, reward_mask):
    seq = seq_samples.astype(jnp.int32).reshape(ROWS * COLS)
    tab = jnp.zeros((VOCAB_PAD,), jnp.float32).at[: reward_mask.shape[0]].set(
        reward_mask.astype(jnp.float32)
    )
    tab2d = jnp.broadcast_to(tab[:, None], (VOCAB_PAD, L))
    return _sc_kernel(seq, tab2d)
